# Initial kernel scaffold; baseline (speedup 1.0000x reference)
import jax
import jax.numpy as jnp
from jax.experimental import pallas as pl


def kernel(x, emb_weight):
    raise NotImplementedError("write your pallas kernel here")


# speedup vs baseline: 5.5431x; 5.5431x over previous
import functools
import math

import jax
import jax.numpy as jnp
from jax import lax
from jax.experimental import pallas as pl
from jax.experimental.pallas import tpu as pltpu
from jax.experimental.pallas import tpu_sc as plsc

_SQRT3 = math.sqrt(3)

_NW = 32
_CH = 12800
_L = 16


@functools.lru_cache(maxsize=None)
def _build(n_total: int):
    assert n_total % (_NW * _CH) == 0
    per_w = n_total // _NW
    n_chunks = per_w // _CH
    n_groups = _CH // _L

    mesh = plsc.VectorSubcoreMesh(core_axis_name="c", subcore_axis_name="s")

    @functools.partial(
        pl.kernel,
        mesh=mesh,
        compiler_params=pltpu.CompilerParams(needs_layout_passes=False),
        out_type=jax.ShapeDtypeStruct((3 * n_total,), jnp.float32),
        scratch_types=[
            pltpu.VMEM((_CH,), jnp.int32),
            pltpu.VMEM((3 * _CH,), jnp.float32),
            pltpu.VMEM((2, 3), jnp.float32),
        ],
    )
    def _emb(x_hbm, w_hbm, out_hbm, x_vmem, out_vmem, w_vmem):
        wid = lax.axis_index("s") * 2 + lax.axis_index("c")
        base = wid * per_w

        pltpu.sync_copy(w_hbm, w_vmem)

        iota = lax.iota(jnp.int32, _L)
        zeros = iota * 0
        koff = [(iota + _L * t) // 3 for t in range(3)]
        dsel = [(iota + _L * t) % 3 for t in range(3)]
        sqrt3 = jnp.full((_L,), _SQRT3, jnp.float32)
        w0 = [plsc.load_gather(w_vmem, [zeros, dsel[t]]) * sqrt3
              for t in range(3)]
        w1 = [plsc.load_gather(w_vmem, [zeros + 1, dsel[t]]) * sqrt3
              for t in range(3)]

        for c in range(n_chunks):
            off = base + c * _CH
            pltpu.sync_copy(x_hbm.at[pl.ds(off, _CH)], x_vmem)

            def body(g, _):
                kbase = g * _L
                obase = g * (3 * _L)
                for t in range(3):
                    xv = plsc.load_gather(x_vmem, [kbase + koff[t]])
                    ov = jnp.where(xv == zeros, w0[t], w1[t])
                    out_vmem[pl.ds(obase + _L * t, _L)] = ov
                return 0

            lax.fori_loop(0, n_groups, body, 0)
            pltpu.sync_copy(out_vmem, out_hbm.at[pl.ds(3 * off, 3 * _CH)])

    return _emb


def kernel(x, emb_weight):
    b0, b1 = x.shape
    n_total = b0 * b1
    x_flat = x.reshape(n_total).astype(jnp.int32)
    out_flat = _build(n_total)(x_flat, emb_weight)
    return out_flat.reshape(b0, b1, 3)


# speedup vs baseline: 5.6956x; 1.0275x over previous
import functools
import math

import jax
import jax.numpy as jnp
from jax import lax
from jax.experimental import pallas as pl
from jax.experimental.pallas import tpu as pltpu
from jax.experimental.pallas import tpu_sc as plsc

_SQRT3 = math.sqrt(3)

_NW = 32
_CH = 10240
_L = 16
_U = 16


@functools.lru_cache(maxsize=None)
def _build(n_total: int):
    assert n_total % (_NW * _CH) == 0
    per_w = n_total // _NW
    n_chunks = per_w // _CH
    n_iters = _CH // (_L * _U)

    mesh = plsc.VectorSubcoreMesh(core_axis_name="c", subcore_axis_name="s")

    @functools.partial(
        pl.kernel,
        mesh=mesh,
        compiler_params=pltpu.CompilerParams(needs_layout_passes=False),
        out_type=jax.ShapeDtypeStruct((3 * n_total,), jnp.float32),
        scratch_types=[
            pltpu.VMEM((2, _CH), jnp.int32),
            pltpu.VMEM((2, 3 * _CH), jnp.float32),
            pltpu.VMEM((2, 3), jnp.float32),
            pltpu.SemaphoreType.DMA,
            pltpu.SemaphoreType.DMA,
            pltpu.SemaphoreType.DMA,
            pltpu.SemaphoreType.DMA,
        ],
    )
    def _emb(x_hbm, w_hbm, out_hbm, x_vmem, out_vmem, w_vmem,
             sin0, sin1, sout0, sout1):
        sin = [sin0, sin1]
        sout = [sout0, sout1]
        wid = lax.axis_index("s") * 2 + lax.axis_index("c")
        base = wid * per_w

        pltpu.sync_copy(w_hbm, w_vmem)

        def vperm(v, idx):
            dnums = lax.GatherDimensionNumbers(
                offset_dims=(), collapsed_slice_dims=(0,),
                start_index_map=(0,))
            return lax.gather(
                v, idx[:, None], dnums, slice_sizes=(1,),
                mode=lax.GatherScatterMode.PROMISE_IN_BOUNDS)

        iota = lax.iota(jnp.int32, _L)
        koff = [(iota + _L * t) // 3 for t in range(3)]
        dsel = [(iota + _L * t) % 3 for t in range(3)]
        zeros = iota * 0
        sqrt3 = jnp.full((_L,), _SQRT3, jnp.float32)
        w0 = [plsc.load_gather(w_vmem, [zeros, dsel[t]]) * sqrt3
              for t in range(3)]
        w1 = [plsc.load_gather(w_vmem, [zeros + 1, dsel[t]]) * sqrt3
              for t in range(3)]

        def copy_in(c):
            return pltpu.async_copy(
                x_hbm.at[pl.ds(base + c * _CH, _CH)],
                x_vmem.at[c % 2], sin[c % 2])

        def copy_out(c):
            return pltpu.async_copy(
                out_vmem.at[c % 2],
                out_hbm.at[pl.ds(3 * (base + c * _CH), 3 * _CH)],
                sout[c % 2])

        in_flight = copy_in(0)
        next_in = None
        out_flight = [None, None]
        for c in range(n_chunks):
            buf = c % 2
            if c + 1 < n_chunks:
                next_in = copy_in(c + 1)
            in_flight.wait()
            if out_flight[buf] is not None:
                out_flight[buf].wait()

            def body(j, _):
                kb = j * (_L * _U)
                ob = 3 * kb
                for u in range(_U):
                    xv = x_vmem[buf, pl.ds(kb + u * _L, _L)]
                    for t in range(3):
                        sv = vperm(xv, koff[t])
                        ov = jnp.where(sv == zeros, w0[t], w1[t])
                        out_vmem[buf, pl.ds(ob + u * 48 + _L * t, _L)] = ov
                return 0

            lax.fori_loop(0, n_iters, body, 0)
            out_flight[buf] = copy_out(c)
            in_flight = next_in
        for f in out_flight:
            if f is not None:
                f.wait()

    return _emb


def kernel(x, emb_weight):
    b0, b1 = x.shape
    n_total = b0 * b1
    x_flat = x.reshape(n_total).astype(jnp.int32)
    out_flat = _build(n_total)(x_flat, emb_weight)
    return out_flat.reshape(b0, b1, 3)
